# baseline (speedup 1.0000x reference)
import functools

import jax
import jax.numpy as jnp
from jax import lax
from jax.experimental import pallas as pl
from jax.experimental.pallas import tpu as pltpu
from jax.experimental.pallas import tpu_sc as plsc

B, L, D = 4096, 200, 64
N = B * L
NC, NS = 2, 16
NW = NC * NS
C = 128
NBUF = 4
OUTER = L // NBUF

_mesh = plsc.VectorSubcoreMesh(core_axis_name="c", subcore_axis_name="s")


@functools.partial(
    pl.kernel,
    mesh=_mesh,
    out_type=jax.ShapeDtypeStruct((L, 8, NW, 1024), jnp.float32),
    compiler_params=pltpu.CompilerParams(use_tc_tiling_on_sc=False,
                                         needs_layout_passes=False),
    scratch_types=(
        [pltpu.VMEM((C,), jnp.int32) for _ in range(NBUF)]
        + [pltpu.VMEM((C,), jnp.int32) for _ in range(NBUF)]
        + [pltpu.VMEM((C, D), jnp.float32) for _ in range(NBUF)]
        + [pltpu.VMEM((C, D), jnp.float32) for _ in range(NBUF)]
        + [pltpu.VMEM((8, 1024), jnp.float32) for _ in range(2)]
        + [pltpu.SemaphoreType.DMA for _ in range(2 * NBUF + 2)]
    ),
)
def _sol_embedding(t_hbm, v_hbm, tt_hbm, vt_hbm, out_hbm, *scr):
    ti = scr[0:NBUF]
    vi = scr[NBUF:2 * NBUF]
    bufv = scr[2 * NBUF:3 * NBUF]
    buft = scr[3 * NBUF:4 * NBUF]
    slab = scr[4 * NBUF:4 * NBUF + 2]
    semi = scr[4 * NBUF + 2:5 * NBUF + 2]
    semg = scr[5 * NBUF + 2:6 * NBUF + 2]
    semslab = scr[6 * NBUF + 2:6 * NBUF + 4]

    wid = lax.axis_index("s") * NC + lax.axis_index("c")

    def issue_idx(l, s):
        off = wid * C
        pltpu.async_copy(t_hbm.at[l, pl.ds(off, C)], ti[s], semi[s])
        pltpu.async_copy(v_hbm.at[l, pl.ds(off, C)], vi[s], semi[s])

    def wait_idx(s):
        pltpu.make_async_copy(t_hbm.at[0, pl.ds(0, C)], ti[s], semi[s]).wait()
        pltpu.make_async_copy(v_hbm.at[0, pl.ds(0, C)], vi[s], semi[s]).wait()

    def start_gathers(s):
        pltpu.async_copy(vt_hbm.at[vi[s]], bufv[s], semg[s])
        pltpu.async_copy(tt_hbm.at[ti[s]], buft[s], semg[s])

    def wait_gathers(s):
        pltpu.make_async_copy(vt_hbm.at[vi[s]], bufv[s], semg[s]).wait()
        pltpu.make_async_copy(tt_hbm.at[ti[s]], buft[s], semg[s]).wait()

    def start_slab_write(l, p):
        pltpu.async_copy(slab[p], out_hbm.at[l, :, wid, :], semslab[p])

    def wait_slab_write(p):
        pltpu.make_async_copy(slab[p], out_hbm.at[0, :, 0, :],
                              semslab[p]).wait()

    def compute_slab(s, p):
        bv, bt, sl = bufv[s], buft[s], slab[p]

        def body(r, carry):
            i16 = lax.iota(jnp.int32, 16)
            rowbase = lax.shift_right_logical(i16, 3)
            col = lax.shift_left(lax.bitwise_and(i16, 7), 7)
            colr = col + r
            for g in range(D // 16):
                x = bv[r, pl.ds(16 * g, 16)] + bt[r, pl.ds(16 * g, 16)]
                plsc.store_scatter(sl, [rowbase + 2 * g, colr], x)
            return carry

        lax.fori_loop(0, C, body, 0)

    pltpu.sync_copy(t_hbm.at[0, pl.ds(wid * C, C)], ti[0])
    pltpu.sync_copy(v_hbm.at[0, pl.ds(wid * C, C)], vi[0])
    pltpu.sync_copy(t_hbm.at[1, pl.ds(wid * C, C)], ti[1])
    pltpu.sync_copy(v_hbm.at[1, pl.ds(wid * C, C)], vi[1])
    start_gathers(0)
    start_gathers(1)
    issue_idx(2, 2)
    issue_idx(3, 3)

    def outer(k, carry):
        for b in range(NBUF):
            l = k * NBUF + b
            s2 = (b + 2) % NBUF
            p = b % 2

            def stage_a():
                wait_idx(s2)
                start_gathers(s2)

            if b < 2:
                stage_a()
            else:
                @pl.when(k < OUTER - 1)
                def _():
                    stage_a()

            wait_gathers(b)

            @pl.when(k < OUTER - 1)
            def _():
                issue_idx(l + NBUF, b)

            if b < 2:
                @pl.when(k >= 1)
                def _():
                    wait_slab_write(p)
            else:
                wait_slab_write(p)
            compute_slab(b, p)

            start_slab_write(l, p)
        return carry

    lax.fori_loop(0, OUTER, outer, 0)

    wait_slab_write(0)
    wait_slab_write(1)


def kernel(t, v, type_table, value_table):
    tT = jnp.transpose(t.astype(jnp.int32))
    vT = jnp.transpose(v.astype(jnp.int32))
    vt1d = jax.lax.optimization_barrier(jnp.reshape(value_table, (64000000,)))
    vt2d = jnp.reshape(vt1d, (1000000, 64))
    out4 = _sol_embedding(tT, vT, type_table, vt2d)
    o5 = jnp.reshape(out4, (L, 8, NW, 8, 128))
    o6 = jnp.transpose(o5, (2, 4, 0, 1, 3))
    return jnp.reshape(o6, (B, L, D))


# speedup vs baseline: 1.9039x; 1.9039x over previous
import functools

import jax
import jax.numpy as jnp
from jax import lax
from jax.experimental import pallas as pl
from jax.experimental.pallas import tpu as pltpu
from jax.experimental.pallas import tpu_sc as plsc

B, L, D = 4096, 200, 64
N = B * L
NC, NS = 2, 16
NW = NC * NS
C = 128
NBUF = 4
OUTER = L // NBUF

_mesh = plsc.VectorSubcoreMesh(core_axis_name="c", subcore_axis_name="s")


@functools.partial(
    pl.kernel,
    mesh=_mesh,
    out_type=jax.ShapeDtypeStruct((L, 8, NW, 1024), jnp.float32),
    compiler_params=pltpu.CompilerParams(use_tc_tiling_on_sc=False,
                                         needs_layout_passes=False),
    scratch_types=(
        [pltpu.VMEM((C,), jnp.int32) for _ in range(NBUF)]
        + [pltpu.VMEM((C,), jnp.int32) for _ in range(NBUF)]
        + [pltpu.VMEM((C, D), jnp.float32) for _ in range(NBUF)]
        + [pltpu.VMEM((C, D), jnp.float32) for _ in range(NBUF)]
        + [pltpu.VMEM((8, 1024), jnp.float32) for _ in range(2)]
        + [pltpu.SemaphoreType.DMA for _ in range(2 * NBUF + 2)]
    ),
)
def _sol_embedding(t_hbm, v_hbm, tt_hbm, vt_hbm, out_hbm, *scr):
    ti = scr[0:NBUF]
    vi = scr[NBUF:2 * NBUF]
    bufv = scr[2 * NBUF:3 * NBUF]
    buft = scr[3 * NBUF:4 * NBUF]
    slab = scr[4 * NBUF:4 * NBUF + 2]
    semi = scr[4 * NBUF + 2:5 * NBUF + 2]
    semg = scr[5 * NBUF + 2:6 * NBUF + 2]
    semslab = scr[6 * NBUF + 2:6 * NBUF + 4]

    wid = lax.axis_index("s") * NC + lax.axis_index("c")

    def issue_idx(l, s):
        off = wid * C
        pltpu.async_copy(t_hbm.at[l, pl.ds(off, C)], ti[s], semi[s])
        pltpu.async_copy(v_hbm.at[l, pl.ds(off, C)], vi[s], semi[s])

    def wait_idx(s):
        pltpu.make_async_copy(t_hbm.at[0, pl.ds(0, C)], ti[s], semi[s]).wait()
        pltpu.make_async_copy(v_hbm.at[0, pl.ds(0, C)], vi[s], semi[s]).wait()

    def start_gathers(s):
        pltpu.async_copy(vt_hbm.at[vi[s]], bufv[s], semg[s])
        pltpu.async_copy(tt_hbm.at[ti[s]], buft[s], semg[s])

    def wait_gathers(s):
        pltpu.make_async_copy(vt_hbm.at[vi[s]], bufv[s], semg[s]).wait()
        pltpu.make_async_copy(tt_hbm.at[ti[s]], buft[s], semg[s]).wait()

    def start_slab_write(l, p):
        pltpu.async_copy(slab[p], out_hbm.at[l, :, wid, :], semslab[p])

    def wait_slab_write(p):
        pltpu.make_async_copy(slab[p], out_hbm.at[0, :, 0, :],
                              semslab[p]).wait()

    _gd = lax.GatherDimensionNumbers(offset_dims=(), collapsed_slice_dims=(0,),
                                     start_index_map=(0,))

    def _rot(x, idx):
        return lax.gather(x, idx[:, None], _gd, slice_sizes=(1,),
                          mode=lax.GatherScatterMode.PROMISE_IN_BOUNDS)

    def compute_slab(s, p):
        bv, bt, sl = bufv[s], buft[s], slab[p]

        def body(rg, carry):
            i16 = lax.iota(jnp.int32, 16)
            masks, idxp, idxm = {}, {}, {}
            for d in (1, 2, 4, 8):
                masks[d] = lax.bitwise_and(i16, d) == 0
                idxp[d] = lax.bitwise_and(i16 - d, 15)
                idxm[d] = lax.bitwise_and(i16 + d, 15)
            r0 = rg * 16
            for jg in range(D // 16):
                vs = [bv[r0 + rr, pl.ds(16 * jg, 16)]
                      + bt[r0 + rr, pl.ds(16 * jg, 16)] for rr in range(16)]
                for d in (1, 2, 4, 8):
                    nv = list(vs)
                    for i in range(16):
                        if (i & d) == 0:
                            a, b2 = vs[i], vs[i + d]
                            rp = _rot(b2, idxp[d])
                            rm = _rot(a, idxm[d])
                            nv[i] = jnp.where(masks[d], a, rp)
                            nv[i + d] = jnp.where(masks[d], rm, b2)
                    vs = nv
                for jj in range(16):
                    j = 16 * jg + jj
                    start = pl.multiple_of((j % 8) * 128 + 16 * rg, 16)
                    sl[j // 8, pl.ds(start, 16)] = vs[jj]
            return carry

        lax.fori_loop(0, C // 16, body, 0)

    pltpu.sync_copy(t_hbm.at[0, pl.ds(wid * C, C)], ti[0])
    pltpu.sync_copy(v_hbm.at[0, pl.ds(wid * C, C)], vi[0])
    pltpu.sync_copy(t_hbm.at[1, pl.ds(wid * C, C)], ti[1])
    pltpu.sync_copy(v_hbm.at[1, pl.ds(wid * C, C)], vi[1])
    start_gathers(0)
    start_gathers(1)
    issue_idx(2, 2)
    issue_idx(3, 3)

    def outer(k, carry):
        for b in range(NBUF):
            l = k * NBUF + b
            s2 = (b + 2) % NBUF
            p = b % 2

            def stage_a():
                wait_idx(s2)
                start_gathers(s2)

            if b < 2:
                stage_a()
            else:
                @pl.when(k < OUTER - 1)
                def _():
                    stage_a()

            wait_gathers(b)

            @pl.when(k < OUTER - 1)
            def _():
                issue_idx(l + NBUF, b)

            if b < 2:
                @pl.when(k >= 1)
                def _():
                    wait_slab_write(p)
            else:
                wait_slab_write(p)
            compute_slab(b, p)

            start_slab_write(l, p)
        return carry

    lax.fori_loop(0, OUTER, outer, 0)

    wait_slab_write(0)
    wait_slab_write(1)


def kernel(t, v, type_table, value_table):
    tT = jnp.transpose(t.astype(jnp.int32))
    vT = jnp.transpose(v.astype(jnp.int32))
    vt1d = jax.lax.optimization_barrier(jnp.reshape(value_table, (64000000,)))
    vt2d = jnp.reshape(vt1d, (1000000, 64))
    out4 = _sol_embedding(tT, vT, type_table, vt2d)
    o5 = jnp.reshape(out4, (L, 8, NW, 8, 128))
    o6 = jnp.transpose(o5, (2, 4, 0, 1, 3))
    return jnp.reshape(o6, (B, L, D))


# speedup vs baseline: 1.9784x; 1.0392x over previous
import functools

import jax
import jax.numpy as jnp
from jax import lax
from jax.experimental import pallas as pl
from jax.experimental.pallas import tpu as pltpu
from jax.experimental.pallas import tpu_sc as plsc

B, L, D = 4096, 200, 64
N = B * L
NC, NS = 2, 16
NW = NC * NS
C = 128
NBUF = 4
OUTER = L // NBUF

_mesh = plsc.VectorSubcoreMesh(core_axis_name="c", subcore_axis_name="s")


@functools.partial(
    pl.kernel,
    mesh=_mesh,
    out_type=jax.ShapeDtypeStruct((L, 8, NW, 1024), jnp.float32),
    compiler_params=pltpu.CompilerParams(use_tc_tiling_on_sc=False,
                                         needs_layout_passes=False),
    scratch_types=(
        [pltpu.VMEM((C,), jnp.int32) for _ in range(NBUF)]
        + [pltpu.VMEM((C,), jnp.int32) for _ in range(NBUF)]
        + [pltpu.VMEM((C, 2 * D), jnp.float32) for _ in range(NBUF)]
        + [pltpu.VMEM((C, D), jnp.float32) for _ in range(NBUF)]
        + [pltpu.VMEM((8, 1024), jnp.float32) for _ in range(2)]
        + [pltpu.SemaphoreType.DMA for _ in range(2 * NBUF + 2)]
    ),
)
def _sol_embedding(t_hbm, v_hbm, tt_hbm, vt_hbm, out_hbm, *scr):
    ti = scr[0:NBUF]
    vi = scr[NBUF:2 * NBUF]
    bufv = scr[2 * NBUF:3 * NBUF]
    buft = scr[3 * NBUF:4 * NBUF]
    slab = scr[4 * NBUF:4 * NBUF + 2]
    semi = scr[4 * NBUF + 2:5 * NBUF + 2]
    semg = scr[5 * NBUF + 2:6 * NBUF + 2]
    semslab = scr[6 * NBUF + 2:6 * NBUF + 4]

    wid = lax.axis_index("s") * NC + lax.axis_index("c")

    def issue_idx(l, s):
        off = wid * C
        pltpu.async_copy(t_hbm.at[l, pl.ds(off, C)], ti[s], semi[s])
        pltpu.async_copy(v_hbm.at[l, pl.ds(off, C)], vi[s], semi[s])

    def wait_idx(s):
        pltpu.make_async_copy(t_hbm.at[0, pl.ds(0, C)], ti[s], semi[s]).wait()
        pltpu.make_async_copy(v_hbm.at[0, pl.ds(0, C)], vi[s], semi[s]).wait()

    def start_gathers(s):
        pltpu.async_copy(vt_hbm.at[vi[s]], bufv[s], semg[s])
        pltpu.async_copy(tt_hbm.at[ti[s]], buft[s], semg[s])

    def wait_gathers(s):
        pltpu.make_async_copy(vt_hbm.at[vi[s]], bufv[s], semg[s]).wait()
        pltpu.make_async_copy(tt_hbm.at[ti[s]], buft[s], semg[s]).wait()

    def start_slab_write(l, p):
        pltpu.async_copy(slab[p], out_hbm.at[l, :, wid, :], semslab[p])

    def wait_slab_write(p):
        pltpu.make_async_copy(slab[p], out_hbm.at[0, :, 0, :],
                              semslab[p]).wait()

    _gd = lax.GatherDimensionNumbers(offset_dims=(), collapsed_slice_dims=(0,),
                                     start_index_map=(0,))

    def _rot(x, idx):
        return lax.gather(x, idx[:, None], _gd, slice_sizes=(1,),
                          mode=lax.GatherScatterMode.PROMISE_IN_BOUNDS)

    def compute_slab(s, p):
        bv, bt, sl = bufv[s], buft[s], slab[p]

        def body(rg, carry):
            i16 = lax.iota(jnp.int32, 16)
            masks, idxp, idxm = {}, {}, {}
            for d in (1, 2, 4, 8):
                masks[d] = lax.bitwise_and(i16, d) == 0
                idxp[d] = lax.bitwise_and(i16 - d, 15)
                idxm[d] = lax.bitwise_and(i16 + d, 15)
            r0 = rg * 16
            for jg in range(D // 16):
                vs = [bv[r0 + rr, pl.ds(16 * jg, 16)]
                      + bt[r0 + rr, pl.ds(16 * jg, 16)] for rr in range(16)]
                for d in (1, 2, 4, 8):
                    nv = list(vs)
                    for i in range(16):
                        if (i & d) == 0:
                            a, b2 = vs[i], vs[i + d]
                            rp = _rot(b2, idxp[d])
                            rm = _rot(a, idxm[d])
                            nv[i] = jnp.where(masks[d], a, rp)
                            nv[i + d] = jnp.where(masks[d], rm, b2)
                    vs = nv
                for jj in range(16):
                    j = 16 * jg + jj
                    start = pl.multiple_of((j % 8) * 128 + 16 * rg, 16)
                    sl[j // 8, pl.ds(start, 16)] = vs[jj]
            return carry

        lax.fori_loop(0, C // 16, body, 0)

    pltpu.sync_copy(t_hbm.at[0, pl.ds(wid * C, C)], ti[0])
    pltpu.sync_copy(v_hbm.at[0, pl.ds(wid * C, C)], vi[0])
    pltpu.sync_copy(t_hbm.at[1, pl.ds(wid * C, C)], ti[1])
    pltpu.sync_copy(v_hbm.at[1, pl.ds(wid * C, C)], vi[1])
    start_gathers(0)
    start_gathers(1)
    issue_idx(2, 2)
    issue_idx(3, 3)

    def outer(k, carry):
        for b in range(NBUF):
            l = k * NBUF + b
            s2 = (b + 2) % NBUF
            p = b % 2

            def stage_a():
                wait_idx(s2)
                start_gathers(s2)

            if b < 2:
                stage_a()
            else:
                @pl.when(k < OUTER - 1)
                def _():
                    stage_a()

            wait_gathers(b)

            @pl.when(k < OUTER - 1)
            def _():
                issue_idx(l + NBUF, b)

            if b < 2:
                @pl.when(k >= 1)
                def _():
                    wait_slab_write(p)
            else:
                wait_slab_write(p)
            compute_slab(b, p)

            start_slab_write(l, p)
        return carry

    lax.fori_loop(0, OUTER, outer, 0)

    wait_slab_write(0)
    wait_slab_write(1)


def kernel(t, v, type_table, value_table):
    tT = jnp.transpose(t.astype(jnp.int32))
    vT = jnp.transpose(v.astype(jnp.int32))
    vt128 = jnp.pad(value_table, ((0, 0), (0, D)))
    out4 = _sol_embedding(tT, vT, type_table, vt128)
    o5 = jnp.reshape(out4, (L, 8, NW, 8, 128))
    o6 = jnp.transpose(o5, (2, 4, 0, 1, 3))
    return jnp.reshape(o6, (B, L, D))
